# baseline (speedup 1.0000x reference)
import functools

import jax
import jax.numpy as jnp
from jax import lax
from jax.experimental import pallas as pl
from jax.experimental.pallas import tpu as pltpu
from jax.experimental.pallas import tpu_sc as plsc

_GATHER_DNUMS = lax.GatherDimensionNumbers(
    offset_dims=(), collapsed_slice_dims=(0,), start_index_map=(0,))


def _bcast_lane(v16, n):
    return lax.gather(v16, jnp.full((16, 1), n, jnp.int32), _GATHER_DNUMS,
                      slice_sizes=(1,),
                      mode=lax.GatherScatterMode.PROMISE_IN_BOUNDS)


_B = 16384
_J = 200
_D = 64
_NW = 32
_CI = 4096
_NIC = _B // _CI
_KT = _D // 8
_G = _CI // 16


def _sc_embed(idx_hbm, w_hbm, out_hbm,
              idx_v, tf_v, buf0, buf1, buf2, w_v, w0s, dws,
              sem_i, sem_o0, sem_o1, sem_o2):
    wid = lax.axis_index("s") * 2 + lax.axis_index("c")
    m_per_w = _J * _NIC // _NW
    m_lo = wid * m_per_w
    m_hi = m_lo + m_per_w
    out_bufs = (buf0, buf1, buf2)
    sem_o = (sem_o0, sem_o1, sem_o2)

    pltpu.sync_copy(w_hbm, w_v)
    for k in range(_D):
        a = _bcast_lane(w_v[pl.ds((k // 16) * 16, 16)], k % 16)
        b = _bcast_lane(w_v[pl.ds(_D + (k // 16) * 16, 16)], k % 16)
        w0s.at[k][:] = a
        dws.at[k][:] = b - a

    def in_copy(m):
        j, ic = m // _NIC, m % _NIC
        return pltpu.make_async_copy(
            idx_hbm.at[j, pl.ds(ic * _CI, _CI)], idx_v, sem_i)

    def out_copy(m, kt, b):
        j, ic = m // _NIC, m % _NIC
        return pltpu.make_async_copy(
            out_bufs[b],
            out_hbm.at[j, pl.ds(kt * 8, 8), pl.ds(ic * _CI, _CI)],
            sem_o[b])

    def build_tf():
        @plsc.parallel_loop(0, _G, unroll=4)
        def cvt(g):
            tf_v[pl.ds(g * 16, 16)] = (
                idx_v[pl.ds(g * 16, 16)].astype(jnp.float32))

    def macro(m, first):
        in_copy(m).wait()
        build_tf()
        in_copy(jnp.minimum(m + 1, m_hi - 1)).start()
        for kt in range(_KT):
            b = kt % 3
            if not (first and kt < 3):
                out_copy(m, kt, b).wait()
            compute_k(m, kt, b)

    def compute_k(m, kt, b):
        buf = out_bufs[b]

        @plsc.parallel_loop(0, 8, unroll=1)
        def row(k):
            w0k = w0s.at[kt * 8 + k][:]
            dwk = dws.at[kt * 8 + k][:]
            brow = buf.at[k]

            @plsc.parallel_loop(0, _G, unroll=8)
            def seg(g):
                tf = tf_v[pl.ds(g * 16, 16)]
                brow[pl.ds(g * 16, 16)] = w0k + tf * dwk

        out_copy(m, kt, b).start()

    in_copy(m_lo).start()
    macro(m_lo, first=True)

    def steady(m, carry):
        macro(m, first=False)
        return carry

    lax.fori_loop(m_lo + 1, m_hi, steady, 0)
    out_copy(m_hi - 1, _KT - 3, (_KT - 3) % 3).wait()
    out_copy(m_hi - 1, _KT - 2, (_KT - 2) % 3).wait()
    out_copy(m_hi - 1, _KT - 1, (_KT - 1) % 3).wait()
    in_copy(m_hi - 1).wait()


def kernel(atom_types, embedding_weight):
    idx_t = atom_types.astype(jnp.int32).T
    wflat = embedding_weight.reshape(-1).astype(jnp.float32)
    mesh = plsc.VectorSubcoreMesh(core_axis_name="c", subcore_axis_name="s")
    run = functools.partial(
        pl.kernel,
        mesh=mesh,
        out_type=jax.ShapeDtypeStruct((_J, _D, _B), jnp.float32),
        compiler_params=pltpu.CompilerParams(use_tc_tiling_on_sc=True),
        scratch_types=[
            pltpu.VMEM((_CI,), jnp.int32),
            pltpu.VMEM((_CI,), jnp.float32),
            pltpu.VMEM((8, _CI), jnp.float32),
            pltpu.VMEM((8, _CI), jnp.float32),
            pltpu.VMEM((8, _CI), jnp.float32),
            pltpu.VMEM((2 * _D,), jnp.float32),
            pltpu.VMEM((_D, 16), jnp.float32),
            pltpu.VMEM((_D, 16), jnp.float32),
            pltpu.SemaphoreType.DMA,
            pltpu.SemaphoreType.DMA,
            pltpu.SemaphoreType.DMA,
            pltpu.SemaphoreType.DMA,
        ],
    )(_sc_embed)
    out = run(idx_t, wflat)
    return out.transpose(2, 0, 1)


# speedup vs baseline: 1.0022x; 1.0022x over previous
import functools

import jax
import jax.numpy as jnp
from jax import lax
from jax.experimental import pallas as pl
from jax.experimental.pallas import tpu as pltpu
from jax.experimental.pallas import tpu_sc as plsc

_GATHER_DNUMS = lax.GatherDimensionNumbers(
    offset_dims=(), collapsed_slice_dims=(0,), start_index_map=(0,))


def _bcast_lane(v16, n):
    return lax.gather(v16, jnp.full((16, 1), n, jnp.int32), _GATHER_DNUMS,
                      slice_sizes=(1,),
                      mode=lax.GatherScatterMode.PROMISE_IN_BOUNDS)


_B = 16384
_J = 200
_D = 64
_NW = 32
_CI = 4096
_NIC = _B // _CI
_KT = _D // 8
_G = _CI // 16


def _sc_embed(idx_hbm, w_hbm, out_hbm,
              idx_v, tf_v, buf0, buf1, w_v, w0s, dws,
              sem_i, sem_o0, sem_o1):
    wid = lax.axis_index("s") * 2 + lax.axis_index("c")
    m_per_w = _J * _NIC // _NW
    m_lo = wid * m_per_w
    m_hi = m_lo + m_per_w
    out_bufs = (buf0, buf1)
    sem_o = (sem_o0, sem_o1)

    pltpu.sync_copy(w_hbm, w_v)
    for k in range(_D):
        a = _bcast_lane(w_v[pl.ds((k // 16) * 16, 16)], k % 16)
        b = _bcast_lane(w_v[pl.ds(_D + (k // 16) * 16, 16)], k % 16)
        w0s.at[k][:] = a
        dws.at[k][:] = b - a

    def in_copy(m):
        j, ic = m // _NIC, m % _NIC
        return pltpu.make_async_copy(
            idx_hbm.at[j, pl.ds(ic * _CI, _CI)], idx_v, sem_i)

    def out_copy(m, kt, b):
        j, ic = m // _NIC, m % _NIC
        return pltpu.make_async_copy(
            out_bufs[b],
            out_hbm.at[j, pl.ds(kt * 8, 8), pl.ds(ic * _CI, _CI)],
            sem_o[b])

    def build_tf():
        @plsc.parallel_loop(0, _G, unroll=4)
        def cvt(g):
            tf_v[pl.ds(g * 16, 16)] = (
                idx_v[pl.ds(g * 16, 16)].astype(jnp.float32))

    def macro(m, first):
        in_copy(m).wait()
        build_tf()
        in_copy(jnp.minimum(m + 1, m_hi - 1)).start()
        for kt in range(_KT):
            b = kt % 2
            if not (first and kt < 2):
                out_copy(m, kt, b).wait()
            compute_k(m, kt, b)

    def compute_k(m, kt, b):
        buf = out_bufs[b]

        @plsc.parallel_loop(0, 8, unroll=1)
        def row(k):
            w0k = w0s.at[kt * 8 + k][:]
            dwk = dws.at[kt * 8 + k][:]
            brow = buf.at[k]

            @plsc.parallel_loop(0, _G, unroll=8)
            def seg(g):
                tf = tf_v[pl.ds(g * 16, 16)]
                brow[pl.ds(g * 16, 16)] = w0k + tf * dwk

        out_copy(m, kt, b).start()

    in_copy(m_lo).start()
    macro(m_lo, first=True)

    def steady(m, carry):
        macro(m, first=False)
        return carry

    lax.fori_loop(m_lo + 1, m_hi, steady, 0)
    out_copy(m_hi - 1, _KT - 2, 0).wait()
    out_copy(m_hi - 1, _KT - 1, 1).wait()
    in_copy(m_hi - 1).wait()


def kernel(atom_types, embedding_weight):
    idx_t = atom_types.astype(jnp.int32).T
    wflat = embedding_weight.reshape(-1).astype(jnp.float32)
    mesh = plsc.VectorSubcoreMesh(core_axis_name="c", subcore_axis_name="s")
    run = functools.partial(
        pl.kernel,
        mesh=mesh,
        out_type=jax.ShapeDtypeStruct((_J, _D, _B), jnp.float32),
        compiler_params=pltpu.CompilerParams(use_tc_tiling_on_sc=True),
        scratch_types=[
            pltpu.VMEM((_CI,), jnp.int32),
            pltpu.VMEM((_CI,), jnp.float32),
            pltpu.VMEM((8, _CI), jnp.float32),
            pltpu.VMEM((8, _CI), jnp.float32),
            pltpu.VMEM((2 * _D,), jnp.float32),
            pltpu.VMEM((_D, 16), jnp.float32),
            pltpu.VMEM((_D, 16), jnp.float32),
            pltpu.SemaphoreType.DMA,
            pltpu.SemaphoreType.DMA,
            pltpu.SemaphoreType.DMA,
        ],
    )(_sc_embed)
    out = run(idx_t, wflat)
    return out.transpose(2, 0, 1)
